# Initial kernel scaffold; baseline (speedup 1.0000x reference)
import functools

import jax
import jax.numpy as jnp
from jax import lax
from jax.experimental import pallas as pl
from jax.experimental.pallas import tpu as pltpu
from jax.experimental.pallas import tpu_sc as plsc

_N = 10000
_E = 320000
_H = 128
_ED = 16
_L = 3
_G = 64

_NC = 2
_NS = 16
_NW = _NC * _NS
_EPW = _E // _NW
_K = 80
_NCHUNK = _EPW // _K
_RPT = _N // _NS
_RC = 125


def _sc_body(h_hbm, ec_hbm, src_hbm, dst_hbm, out_hbm,
             aggr_sh, srcv, dstv, hbuf, ebuf, zbuf, sem):
    cid = lax.axis_index("c")
    sid = lax.axis_index("s")
    wid = sid * _NC + cid

    def _zrow(i, _):
        for k in range(_H // 16):
            zbuf[i, pl.ds(k * 16, 16)] = jnp.zeros((16,), jnp.float32)
        return 0
    lax.fori_loop(0, _RC, _zrow, 0)
    row0 = sid * _RPT
    for j in range(_RPT // _RC):
        pltpu.sync_copy(zbuf, aggr_sh.at[pl.ds(row0 + j * _RC, _RC)])
    plsc.subcore_barrier()

    ebase = wid * _EPW

    def _chunk(c, _):
        base = ebase + c * _K
        pltpu.sync_copy(src_hbm.at[pl.ds(base, _K)], srcv)
        pltpu.sync_copy(dst_hbm.at[pl.ds(base, _K)], dstv)
        pltpu.async_copy(h_hbm.at[srcv], hbuf, sem).wait()
        pltpu.sync_copy(ec_hbm.at[pl.ds(base, _K)], ebuf)

        def _row(i, _):
            for k in range(_H // 16):
                s = pl.ds(k * 16, 16)
                ebuf[i, s] = jnp.maximum(hbuf[i, s] + ebuf[i, s], 0.0)
            return 0
        lax.fori_loop(0, _K, _row, 0)
        pltpu.sync_copy(ebuf, aggr_sh.at[dstv], add=True)
        return 0

    lax.fori_loop(0, _NCHUNK, _chunk, 0)
    plsc.subcore_barrier()

    for j in range(_RPT // _RC):
        r = row0 + j * _RC
        pltpu.sync_copy(aggr_sh.at[pl.ds(r, _RC)], zbuf)
        pltpu.sync_copy(zbuf, out_hbm.at[cid, pl.ds(r, _RC)])


_sc_msg = pl.kernel(
    _sc_body,
    out_type=jax.ShapeDtypeStruct((_NC, _N, _H), jnp.float32),
    mesh=plsc.VectorSubcoreMesh(core_axis_name="c", subcore_axis_name="s"),
    scratch_types=[
        pltpu.VMEM_SHARED((_N, _H), jnp.float32),
        pltpu.VMEM((_K,), jnp.int32),
        pltpu.VMEM((_K,), jnp.int32),
        pltpu.VMEM((_K, _H), jnp.float32),
        pltpu.VMEM((_K, _H), jnp.float32),
        pltpu.VMEM((_RC, _H), jnp.float32),
        pltpu.SemaphoreType.DMA,
    ],
)


_BE = 8000


def _ec_body(ea_ref, w_ref, b_ref, out_ref):
    out_ref[...] = jnp.dot(ea_ref[...], w_ref[...],
                           preferred_element_type=jnp.float32) + b_ref[...]


_ec_call = pl.pallas_call(
    _ec_body,
    grid=(_E // _BE,),
    in_specs=[
        pl.BlockSpec((_BE, _ED), lambda i: (i, 0)),
        pl.BlockSpec((_ED, _H), lambda i: (0, 0)),
        pl.BlockSpec((1, _H), lambda i: (0, 0)),
    ],
    out_specs=pl.BlockSpec((_BE, _H), lambda i: (i, 0)),
    out_shape=jax.ShapeDtypeStruct((_E, _H), jnp.float32),
)

_BR = 2000


def _mlp_body(h_ref, p_ref, w1_ref, b1_ref, w2_ref, b2_ref, out_ref):
    z = h_ref[...] + p_ref[0] + p_ref[1]
    t = jnp.maximum(jnp.dot(z, w1_ref[...],
                            preferred_element_type=jnp.float32) + b1_ref[...], 0.0)
    out_ref[...] = jnp.maximum(
        jnp.dot(t, w2_ref[...], preferred_element_type=jnp.float32)
        + b2_ref[...], 0.0)


_mlp_call = pl.pallas_call(
    _mlp_body,
    grid=(_N // _BR,),
    in_specs=[
        pl.BlockSpec((_BR, _H), lambda i: (i, 0)),
        pl.BlockSpec((_NC, _BR, _H), lambda i: (0, i, 0)),
        pl.BlockSpec((_H, _H), lambda i: (0, 0)),
        pl.BlockSpec((1, _H), lambda i: (0, 0)),
        pl.BlockSpec((_H, _H), lambda i: (0, 0)),
        pl.BlockSpec((1, _H), lambda i: (0, 0)),
    ],
    out_specs=pl.BlockSpec((_BR, _H), lambda i: (i, 0)),
    out_shape=jax.ShapeDtypeStruct((_N, _H), jnp.float32),
)


def _pool_body(h_ref, bt_ref, wl1_ref, bl1_ref, wl2_ref, bl2_ref, out_ref):
    bt = bt_ref[...]
    gid = lax.broadcasted_iota(jnp.float32, (_N, _G), 1)
    p = jnp.where(bt == gid, 1.0, 0.0)
    sums = lax.dot_general(p, h_ref[...], (((0,), (0,)), ((), ())),
                           preferred_element_type=jnp.float32)
    counts = jnp.sum(p, axis=0)
    g = sums / jnp.maximum(counts, 1.0)[:, None]
    t = jnp.maximum(jnp.dot(g, wl1_ref[...],
                            preferred_element_type=jnp.float32) + bl1_ref[...],
                    0.0)
    out_ref[...] = (jnp.dot(t, wl2_ref[...],
                            preferred_element_type=jnp.float32) + bl2_ref[...])


_pool_call = pl.pallas_call(
    _pool_body,
    out_shape=jax.ShapeDtypeStruct((_G, 1), jnp.float32),
)


def kernel(x, edge_index, batch, edge_attr, We, be, W1, b1, W2, b2,
           Wl1, bl1, Wl2, bl2):
    src = edge_index[0].astype(jnp.int32)
    dst = edge_index[1].astype(jnp.int32)
    batf = batch.astype(jnp.float32).reshape(_N, 1)
    h = x
    for l in range(_L):
        ec = _ec_call(edge_attr, We[l], be[l].reshape(1, _H))
        parts = _sc_msg(h, ec, src, dst)
        h = _mlp_call(h, parts, W1[l], b1[l].reshape(1, _H),
                      W2[l], b2[l].reshape(1, _H))
    out2 = _pool_call(h, batf, Wl1, bl1.reshape(1, _H),
                      Wl2, bl2.reshape(1, 1))
    return out2.reshape(-1)


# speedup vs baseline: 2.5612x; 2.5612x over previous
import functools

import jax
import jax.numpy as jnp
from jax import lax
from jax.experimental import pallas as pl
from jax.experimental.pallas import tpu as pltpu
from jax.experimental.pallas import tpu_sc as plsc

_N = 10000
_E = 320000
_H = 128
_ED = 16
_L = 3
_G = 64

_NC = 2
_NS = 16
_NW = _NC * _NS
_EPW = _E // _NW
_K = 80
_NCHUNK = _EPW // _K
_NPAD = 10240
_RPT = _NPAD // _NS
_RC = 128


def _sc_body(h_hbm, ec_hbm, src_hbm, dst_hbm, out_hbm,
             aggr_sh, srcv, dstv, hbuf, ebuf, zbuf, sem):
    cid = lax.axis_index("c")
    sid = lax.axis_index("s")
    wid = sid * _NC + cid

    def _zrow(i, _):
        for k in range(_H // 16):
            zbuf[i, pl.ds(k * 16, 16)] = jnp.zeros((16,), jnp.float32)
        return 0
    lax.fori_loop(0, _RC, _zrow, 0)
    row0 = sid * _RPT
    for j in range(_RPT // _RC):
        pltpu.sync_copy(zbuf, aggr_sh.at[pl.ds(row0 + j * _RC, _RC)])
    plsc.subcore_barrier()

    ebase = wid * _EPW

    def _chunk(c, _):
        base = ebase + c * _K
        pltpu.sync_copy(src_hbm.at[pl.ds(base, _K)], srcv)
        pltpu.sync_copy(dst_hbm.at[pl.ds(base, _K)], dstv)
        pltpu.async_copy(h_hbm.at[srcv], hbuf, sem).wait()
        pltpu.sync_copy(ec_hbm.at[pl.ds(base, _K)], ebuf)

        def _row(i, _):
            for k in range(_H // 16):
                s = pl.ds(k * 16, 16)
                ebuf[i, s] = jnp.maximum(hbuf[i, s] + ebuf[i, s], 0.0)
            return 0
        lax.fori_loop(0, _K, _row, 0)
        pltpu.sync_copy(ebuf, aggr_sh.at[dstv], add=True)
        return 0

    lax.fori_loop(0, _NCHUNK, _chunk, 0)
    plsc.subcore_barrier()

    for j in range(_RPT // _RC):
        r = row0 + j * _RC
        pltpu.sync_copy(aggr_sh.at[pl.ds(r, _RC)], zbuf)
        pltpu.sync_copy(zbuf, out_hbm.at[cid, pl.ds(r, _RC)])


_sc_msg = pl.kernel(
    _sc_body,
    out_type=jax.ShapeDtypeStruct((_NC, _NPAD, _H), jnp.float32),
    mesh=plsc.VectorSubcoreMesh(core_axis_name="c", subcore_axis_name="s"),
    scratch_types=[
        pltpu.VMEM_SHARED((_NPAD, _H), jnp.float32),
        pltpu.VMEM((_K,), jnp.int32),
        pltpu.VMEM((_K,), jnp.int32),
        pltpu.VMEM((_K, _H), jnp.float32),
        pltpu.VMEM((_K, _H), jnp.float32),
        pltpu.VMEM((_RC, _H), jnp.float32),
        pltpu.SemaphoreType.DMA,
    ],
)


_BE = 8000


def _ec_body(ea_ref, w_ref, b_ref, out_ref):
    out_ref[...] = jnp.dot(ea_ref[...], w_ref[...],
                           preferred_element_type=jnp.float32) + b_ref[...]


_ec_call = pl.pallas_call(
    _ec_body,
    grid=(_E // _BE,),
    in_specs=[
        pl.BlockSpec((_BE, _ED), lambda i: (i, 0)),
        pl.BlockSpec((_ED, _H), lambda i: (0, 0)),
        pl.BlockSpec((1, _H), lambda i: (0, 0)),
    ],
    out_specs=pl.BlockSpec((_BE, _H), lambda i: (i, 0)),
    out_shape=jax.ShapeDtypeStruct((_E, _H), jnp.float32),
)

_BR = 2000


def _mlp_body(h_ref, p_ref, w1_ref, b1_ref, w2_ref, b2_ref, out_ref):
    z = h_ref[...] + p_ref[0] + p_ref[1]
    t = jnp.maximum(jnp.dot(z, w1_ref[...],
                            preferred_element_type=jnp.float32) + b1_ref[...], 0.0)
    out_ref[...] = jnp.maximum(
        jnp.dot(t, w2_ref[...], preferred_element_type=jnp.float32)
        + b2_ref[...], 0.0)


_mlp_call = pl.pallas_call(
    _mlp_body,
    grid=(_N // _BR,),
    in_specs=[
        pl.BlockSpec((_BR, _H), lambda i: (i, 0)),
        pl.BlockSpec((_NC, _BR, _H), lambda i: (0, i, 0)),
        pl.BlockSpec((_H, _H), lambda i: (0, 0)),
        pl.BlockSpec((1, _H), lambda i: (0, 0)),
        pl.BlockSpec((_H, _H), lambda i: (0, 0)),
        pl.BlockSpec((1, _H), lambda i: (0, 0)),
    ],
    out_specs=pl.BlockSpec((_BR, _H), lambda i: (i, 0)),
    out_shape=jax.ShapeDtypeStruct((_N, _H), jnp.float32),
)


def _pool_body(h_ref, bt_ref, wl1_ref, bl1_ref, wl2_ref, bl2_ref, out_ref):
    bt = bt_ref[...]
    gid = lax.broadcasted_iota(jnp.int32, (_N, _G), 1)
    p = jnp.where(bt == gid, 1.0, 0.0)
    sums = lax.dot_general(p, h_ref[...], (((0,), (0,)), ((), ())),
                           preferred_element_type=jnp.float32)
    counts = jnp.sum(p, axis=0)
    g = sums / jnp.maximum(counts, 1.0)[:, None]
    t = jnp.maximum(jnp.dot(g, wl1_ref[...],
                            preferred_element_type=jnp.float32) + bl1_ref[...],
                    0.0)
    out_ref[...] = (jnp.dot(t, wl2_ref[...],
                            preferred_element_type=jnp.float32) + bl2_ref[...])


_pool_call = pl.pallas_call(
    _pool_body,
    out_shape=jax.ShapeDtypeStruct((_G, 1), jnp.float32),
)


def kernel(x, edge_index, batch, edge_attr, We, be, W1, b1, W2, b2,
           Wl1, bl1, Wl2, bl2):
    src = edge_index[0].astype(jnp.int32)
    dst = edge_index[1].astype(jnp.int32)
    batf = batch.astype(jnp.int32).reshape(_N, 1)
    h = x
    for l in range(_L):
        ec = _ec_call(edge_attr, We[l], be[l].reshape(1, _H))
        parts = _sc_msg(h, ec, src, dst)
        h = _mlp_call(h, parts, W1[l], b1[l].reshape(1, _H),
                      W2[l], b2[l].reshape(1, _H))
    out2 = _pool_call(h, batf, Wl1, bl1.reshape(1, _H),
                      Wl2, bl2.reshape(1, 1))
    return out2.reshape(-1)


# speedup vs baseline: 3.5230x; 1.3756x over previous
import functools

import jax
import jax.numpy as jnp
from jax import lax
from jax.experimental import pallas as pl
from jax.experimental.pallas import tpu as pltpu
from jax.experimental.pallas import tpu_sc as plsc

_N = 10000
_E = 320000
_H = 128
_ED = 16
_L = 3
_G = 64

_NC = 2
_NS = 16
_NW = _NC * _NS
_EPW = _E // _NW
_K = 40
_NCHUNK = _EPW // _K
_NPAD = 10240
_RPT = _NPAD // _NS


def _sc_body(h_hbm, ec_hbm, src_hbm, dst_hbm, out_hbm,
             aggr_sh, hbuf0, hbuf1, ebuf0, ebuf1,
             sv0, sv1, sv2, sv3, dv0, dv1, dv2, dv3,
             sg0, sg1, se0, se1, ss0, ss1, si0, si1, si2, si3):
    cid = lax.axis_index("c")
    sid = lax.axis_index("s")
    wid = sid * _NC + cid
    ebase = wid * _EPW

    hbufs = (hbuf0, hbuf1)
    ebufs = (ebuf0, ebuf1)
    srcvs = (sv0, sv1, sv2, sv3)
    dstvs = (dv0, dv1, dv2, dv3)
    sgs = (sg0, sg1)
    ses = (se0, se1)
    sss = (ss0, ss1)
    sis = (si0, si1, si2, si3)

    def _zrow(i, _):
        for k in range(_H // 16):
            ebuf0[i, pl.ds(k * 16, 16)] = jnp.zeros((16,), jnp.float32)
        return 0
    lax.fori_loop(0, _K, _zrow, 0)
    row0 = sid * _RPT
    for j in range(_RPT // _K):
        pltpu.sync_copy(ebuf0, aggr_sh.at[pl.ds(row0 + j * _K, _K)])
    plsc.subcore_barrier()

    def _issue_idx(c, q):
        base = ebase + c * _K
        pltpu.async_copy(src_hbm.at[pl.ds(base, _K)], srcvs[q], sis[q])
        pltpu.async_copy(dst_hbm.at[pl.ds(base, _K)], dstvs[q], sis[q])

    def _wait_idx(c, q):
        base = ebase + c * _K
        pltpu.make_async_copy(src_hbm.at[pl.ds(base, _K)], srcvs[q],
                              sis[q]).wait()
        pltpu.make_async_copy(dst_hbm.at[pl.ds(base, _K)], dstvs[q],
                              sis[q]).wait()

    def _issue_gather(slot, q):
        pltpu.async_copy(h_hbm.at[srcvs[q]], hbufs[slot], sgs[slot])

    def _wait_gather(slot, q):
        pltpu.make_async_copy(h_hbm.at[srcvs[q]], hbufs[slot],
                              sgs[slot]).wait()

    def _issue_ec(c, slot):
        pltpu.async_copy(ec_hbm.at[pl.ds(ebase + c * _K, _K)],
                         ebufs[slot], ses[slot])

    def _wait_ec(c, slot):
        pltpu.make_async_copy(ec_hbm.at[pl.ds(ebase + c * _K, _K)],
                              ebufs[slot], ses[slot]).wait()

    def _issue_scatter(slot, q):
        pltpu.async_copy(ebufs[slot], aggr_sh.at[dstvs[q]], sss[slot],
                         add=True)

    def _wait_scatter(slot, q):
        pltpu.make_async_copy(ebufs[slot], aggr_sh.at[dstvs[q]],
                              sss[slot]).wait()

    def _compute(slot):
        hb, eb = hbufs[slot], ebufs[slot]

        def _row(i, _):
            for k in range(_H // 16):
                s = pl.ds(k * 16, 16)
                eb[i, s] = jnp.maximum(hb[i, s] + eb[i, s], 0.0)
            return 0
        lax.fori_loop(0, _K, _row, 0)

    _issue_idx(0, 0)
    _issue_idx(1, 1)
    _issue_idx(2, 2)
    _wait_idx(0, 0)
    _issue_gather(0, 0)
    _issue_ec(0, 0)

    def _step(c, slot, q, prefetch_traced):
        other, q1, q3 = 1 - slot, (q + 1) % 4, (q + 3) % 4
        _wait_gather(slot, q)
        _wait_ec(c, slot)
        _compute(slot)
        _issue_scatter(slot, q)
        _wait_idx(c + 1, q1)
        _issue_gather(other, q1)

        @pl.when(c >= 1)
        def _():
            _wait_scatter(other, q3)
        _issue_ec(c + 1, other)
        if prefetch_traced:
            @pl.when(c + 3 < _NCHUNK)
            def _():
                _issue_idx(c + 3, q3)
        elif c + 3 < _NCHUNK:
            _issue_idx(c + 3, q3)

    def _quad(t, _):
        c0 = t * 4
        _step(c0 + 0, 0, 0, True)
        _step(c0 + 1, 1, 1, True)
        _step(c0 + 2, 0, 2, True)
        _step(c0 + 3, 1, 3, True)
        return 0
    lax.fori_loop(0, (_NCHUNK - 2) // 4, _quad, 0)

    c = _NCHUNK - 2
    _step(c, c % 2, c % 4, False)
    cl = _NCHUNK - 1
    sl, ql = cl % 2, cl % 4
    _wait_gather(sl, ql)
    _wait_ec(cl, sl)
    _compute(sl)
    _issue_scatter(sl, ql)
    _wait_scatter(1 - sl, (ql + 3) % 4)
    _wait_scatter(sl, ql)
    plsc.subcore_barrier()

    for j in range(_RPT // _K):
        r = row0 + j * _K
        pltpu.sync_copy(aggr_sh.at[pl.ds(r, _K)], ebuf0)
        pltpu.sync_copy(ebuf0, out_hbm.at[cid, pl.ds(r, _K)])


_sc_msg = pl.kernel(
    _sc_body,
    out_type=jax.ShapeDtypeStruct((_NC, _NPAD, _H), jnp.float32),
    mesh=plsc.VectorSubcoreMesh(core_axis_name="c", subcore_axis_name="s"),
    scratch_types=(
        [pltpu.VMEM_SHARED((_NPAD, _H), jnp.float32)]
        + [pltpu.VMEM((_K, _H), jnp.float32) for _ in range(4)]
        + [pltpu.VMEM((_K,), jnp.int32) for _ in range(8)]
        + [pltpu.SemaphoreType.DMA for _ in range(10)]
    ),
)


_BE = 8000


def _ec_body(ea_ref, w_ref, b_ref, out_ref):
    out_ref[...] = jnp.dot(ea_ref[...], w_ref[...],
                           preferred_element_type=jnp.float32) + b_ref[...]


_ec_call = pl.pallas_call(
    _ec_body,
    grid=(_E // _BE,),
    in_specs=[
        pl.BlockSpec((_BE, _ED), lambda i: (i, 0)),
        pl.BlockSpec((_ED, _H), lambda i: (0, 0)),
        pl.BlockSpec((1, _H), lambda i: (0, 0)),
    ],
    out_specs=pl.BlockSpec((_BE, _H), lambda i: (i, 0)),
    out_shape=jax.ShapeDtypeStruct((_E, _H), jnp.float32),
)

_BR = 2000


def _mlp_body(h_ref, p_ref, w1_ref, b1_ref, w2_ref, b2_ref, out_ref):
    z = h_ref[...] + p_ref[0] + p_ref[1]
    t = jnp.maximum(jnp.dot(z, w1_ref[...],
                            preferred_element_type=jnp.float32) + b1_ref[...], 0.0)
    out_ref[...] = jnp.maximum(
        jnp.dot(t, w2_ref[...], preferred_element_type=jnp.float32)
        + b2_ref[...], 0.0)


_mlp_call = pl.pallas_call(
    _mlp_body,
    grid=(_N // _BR,),
    in_specs=[
        pl.BlockSpec((_BR, _H), lambda i: (i, 0)),
        pl.BlockSpec((_NC, _BR, _H), lambda i: (0, i, 0)),
        pl.BlockSpec((_H, _H), lambda i: (0, 0)),
        pl.BlockSpec((1, _H), lambda i: (0, 0)),
        pl.BlockSpec((_H, _H), lambda i: (0, 0)),
        pl.BlockSpec((1, _H), lambda i: (0, 0)),
    ],
    out_specs=pl.BlockSpec((_BR, _H), lambda i: (i, 0)),
    out_shape=jax.ShapeDtypeStruct((_N, _H), jnp.float32),
)


def _pool_body(h_ref, bt_ref, wl1_ref, bl1_ref, wl2_ref, bl2_ref, out_ref):
    bt = bt_ref[...]
    gid = lax.broadcasted_iota(jnp.int32, (_N, _G), 1)
    p = jnp.where(bt == gid, 1.0, 0.0)
    sums = lax.dot_general(p, h_ref[...], (((0,), (0,)), ((), ())),
                           preferred_element_type=jnp.float32)
    counts = jnp.sum(p, axis=0)
    g = sums / jnp.maximum(counts, 1.0)[:, None]
    t = jnp.maximum(jnp.dot(g, wl1_ref[...],
                            preferred_element_type=jnp.float32) + bl1_ref[...],
                    0.0)
    out_ref[...] = (jnp.dot(t, wl2_ref[...],
                            preferred_element_type=jnp.float32) + bl2_ref[...])


_pool_call = pl.pallas_call(
    _pool_body,
    out_shape=jax.ShapeDtypeStruct((_G, 1), jnp.float32),
)


def kernel(x, edge_index, batch, edge_attr, We, be, W1, b1, W2, b2,
           Wl1, bl1, Wl2, bl2):
    src = edge_index[0].astype(jnp.int32)
    dst = edge_index[1].astype(jnp.int32)
    batf = batch.astype(jnp.int32).reshape(_N, 1)
    h = x
    for l in range(_L):
        ec = _ec_call(edge_attr, We[l], be[l].reshape(1, _H))
        parts = _sc_msg(h, ec, src, dst)
        h = _mlp_call(h, parts, W1[l], b1[l].reshape(1, _H),
                      W2[l], b2[l].reshape(1, _H))
    out2 = _pool_call(h, batf, Wl1, bl1.reshape(1, _H),
                      Wl2, bl2.reshape(1, 1))
    return out2.reshape(-1)


# speedup vs baseline: 4.2881x; 1.2172x over previous
import functools

import jax
import jax.numpy as jnp
from jax import lax
from jax.experimental import pallas as pl
from jax.experimental.pallas import tpu as pltpu
from jax.experimental.pallas import tpu_sc as plsc

_N = 10000
_E = 320000
_H = 128
_ED = 16
_L = 3
_G = 64

_NC = 2
_NS = 16
_NW = _NC * _NS
_EPW = _E // _NW
_K = 40
_NCHUNK = _EPW // _K
_NPAD = 10240
_RPT = _NPAD // _NS


def _sc_body(h_hbm, ec_hbm, src_hbm, dst_hbm, out_hbm,
             aggr_sh, hbuf0, hbuf1, ebuf0, ebuf1, ebuf2, ebuf3,
             sv0, sv1, sv2, sv3, dv0, dv1, dv2, dv3,
             sg0, sg1, se0, se1, se2, se3,
             ss0, ss1, ss2, ss3, si0, si1, si2, si3):
    cid = lax.axis_index("c")
    sid = lax.axis_index("s")
    wid = sid * _NC + cid
    ebase = wid * _EPW

    hbufs = (hbuf0, hbuf1)
    ebufs = (ebuf0, ebuf1, ebuf2, ebuf3)
    srcvs = (sv0, sv1, sv2, sv3)
    dstvs = (dv0, dv1, dv2, dv3)
    sgs = (sg0, sg1)
    ses = (se0, se1, se2, se3)
    sss = (ss0, ss1, ss2, ss3)
    sis = (si0, si1, si2, si3)

    def _zrow(i, _):
        for k in range(_H // 16):
            ebuf0[i, pl.ds(k * 16, 16)] = jnp.zeros((16,), jnp.float32)
        return 0
    lax.fori_loop(0, _K, _zrow, 0)
    row0 = sid * _RPT
    for j in range(_RPT // _K):
        pltpu.sync_copy(ebuf0, aggr_sh.at[pl.ds(row0 + j * _K, _K)])
    plsc.subcore_barrier()

    def _issue_idx(c, q):
        base = ebase + c * _K
        pltpu.async_copy(src_hbm.at[pl.ds(base, _K)], srcvs[q], sis[q])
        pltpu.async_copy(dst_hbm.at[pl.ds(base, _K)], dstvs[q], sis[q])

    def _wait_idx(c, q):
        base = ebase + c * _K
        pltpu.make_async_copy(src_hbm.at[pl.ds(base, _K)], srcvs[q],
                              sis[q]).wait()
        pltpu.make_async_copy(dst_hbm.at[pl.ds(base, _K)], dstvs[q],
                              sis[q]).wait()

    def _issue_gather(slot, q):
        pltpu.async_copy(h_hbm.at[srcvs[q]], hbufs[slot], sgs[slot])

    def _wait_gather(slot, q):
        pltpu.make_async_copy(h_hbm.at[srcvs[q]], hbufs[slot],
                              sgs[slot]).wait()

    def _issue_ec(c, q):
        pltpu.async_copy(ec_hbm.at[pl.ds(ebase + c * _K, _K)],
                         ebufs[q], ses[q])

    def _wait_ec(c, q):
        pltpu.make_async_copy(ec_hbm.at[pl.ds(ebase + c * _K, _K)],
                              ebufs[q], ses[q]).wait()

    def _issue_scatter(q):
        pltpu.async_copy(ebufs[q], aggr_sh.at[dstvs[q]], sss[q], add=True)

    def _wait_scatter(q):
        pltpu.make_async_copy(ebufs[q], aggr_sh.at[dstvs[q]],
                              sss[q]).wait()

    def _compute(slot, q):
        hb, eb = hbufs[slot], ebufs[q]

        def _row(i, _):
            for k in range(_H // 16):
                s = pl.ds(k * 16, 16)
                eb[i, s] = jnp.maximum(hb[i, s] + eb[i, s], 0.0)
            return 0
        lax.fori_loop(0, _K, _row, 0)

    _issue_idx(0, 0)
    _issue_idx(1, 1)
    _wait_idx(0, 0)
    _issue_gather(0, 0)
    _issue_ec(0, 0)
    _issue_ec(1, 1)

    def _step(c, slot, q, traced):
        other, q1, q2 = 1 - slot, (q + 1) % 4, (q + 2) % 4
        _wait_gather(slot, q)
        _wait_ec(c, q)
        if traced:
            @pl.when(c >= 2)
            def _():
                _wait_scatter(q2)
        elif c >= 2:
            _wait_scatter(q2)
        if traced or c + 2 < _NCHUNK:
            _issue_idx(c + 2, q2)
            _issue_ec(c + 2, q2)
        if traced or c + 1 < _NCHUNK:
            _wait_idx(c + 1, q1)
            _issue_gather(other, q1)
        _compute(slot, q)
        _issue_scatter(q)

    def _quad(t, _):
        c0 = t * 4
        _step(c0 + 0, 0, 0, True)
        _step(c0 + 1, 1, 1, True)
        _step(c0 + 2, 0, 2, True)
        _step(c0 + 3, 1, 3, True)
        return 0
    nloop = (_NCHUNK - 6) // 4
    lax.fori_loop(0, nloop, _quad, 0)
    for c in range(nloop * 4, _NCHUNK):
        _step(c, c % 2, c % 4, False)
    _wait_scatter((_NCHUNK - 2) % 4)
    _wait_scatter((_NCHUNK - 1) % 4)
    plsc.subcore_barrier()

    for j in range(_RPT // _K):
        r = row0 + j * _K
        pltpu.sync_copy(aggr_sh.at[pl.ds(r, _K)], ebuf0)
        pltpu.sync_copy(ebuf0, out_hbm.at[cid, pl.ds(r, _K)])


_sc_msg = pl.kernel(
    _sc_body,
    out_type=jax.ShapeDtypeStruct((_NC, _NPAD, _H), jnp.float32),
    mesh=plsc.VectorSubcoreMesh(core_axis_name="c", subcore_axis_name="s"),
    scratch_types=(
        [pltpu.VMEM_SHARED((_NPAD, _H), jnp.float32)]
        + [pltpu.VMEM((_K, _H), jnp.float32) for _ in range(6)]
        + [pltpu.VMEM((_K,), jnp.int32) for _ in range(8)]
        + [pltpu.SemaphoreType.DMA for _ in range(14)]
    ),
)


_BE = 8000


def _ec_body(ea_ref, w_ref, b_ref, out_ref):
    out_ref[...] = jnp.dot(ea_ref[...], w_ref[...],
                           preferred_element_type=jnp.float32) + b_ref[...]


_ec_call = pl.pallas_call(
    _ec_body,
    grid=(_E // _BE,),
    in_specs=[
        pl.BlockSpec((_BE, _ED), lambda i: (i, 0)),
        pl.BlockSpec((_ED, _H), lambda i: (0, 0)),
        pl.BlockSpec((1, _H), lambda i: (0, 0)),
    ],
    out_specs=pl.BlockSpec((_BE, _H), lambda i: (i, 0)),
    out_shape=jax.ShapeDtypeStruct((_E, _H), jnp.float32),
)

_BR = 2000


def _mlp_body(h_ref, p_ref, w1_ref, b1_ref, w2_ref, b2_ref, out_ref):
    z = h_ref[...] + p_ref[0] + p_ref[1]
    t = jnp.maximum(jnp.dot(z, w1_ref[...],
                            preferred_element_type=jnp.float32) + b1_ref[...], 0.0)
    out_ref[...] = jnp.maximum(
        jnp.dot(t, w2_ref[...], preferred_element_type=jnp.float32)
        + b2_ref[...], 0.0)


_mlp_call = pl.pallas_call(
    _mlp_body,
    grid=(_N // _BR,),
    in_specs=[
        pl.BlockSpec((_BR, _H), lambda i: (i, 0)),
        pl.BlockSpec((_NC, _BR, _H), lambda i: (0, i, 0)),
        pl.BlockSpec((_H, _H), lambda i: (0, 0)),
        pl.BlockSpec((1, _H), lambda i: (0, 0)),
        pl.BlockSpec((_H, _H), lambda i: (0, 0)),
        pl.BlockSpec((1, _H), lambda i: (0, 0)),
    ],
    out_specs=pl.BlockSpec((_BR, _H), lambda i: (i, 0)),
    out_shape=jax.ShapeDtypeStruct((_N, _H), jnp.float32),
)


def _pool_body(h_ref, bt_ref, wl1_ref, bl1_ref, wl2_ref, bl2_ref, out_ref):
    bt = bt_ref[...]
    gid = lax.broadcasted_iota(jnp.int32, (_N, _G), 1)
    p = jnp.where(bt == gid, 1.0, 0.0)
    sums = lax.dot_general(p, h_ref[...], (((0,), (0,)), ((), ())),
                           preferred_element_type=jnp.float32)
    counts = jnp.sum(p, axis=0)
    g = sums / jnp.maximum(counts, 1.0)[:, None]
    t = jnp.maximum(jnp.dot(g, wl1_ref[...],
                            preferred_element_type=jnp.float32) + bl1_ref[...],
                    0.0)
    out_ref[...] = (jnp.dot(t, wl2_ref[...],
                            preferred_element_type=jnp.float32) + bl2_ref[...])


_pool_call = pl.pallas_call(
    _pool_body,
    out_shape=jax.ShapeDtypeStruct((_G, 1), jnp.float32),
)


def kernel(x, edge_index, batch, edge_attr, We, be, W1, b1, W2, b2,
           Wl1, bl1, Wl2, bl2):
    src = edge_index[0].astype(jnp.int32)
    dst = edge_index[1].astype(jnp.int32)
    batf = batch.astype(jnp.int32).reshape(_N, 1)
    h = x
    for l in range(_L):
        ec = _ec_call(edge_attr, We[l], be[l].reshape(1, _H))
        parts = _sc_msg(h, ec, src, dst)
        h = _mlp_call(h, parts, W1[l], b1[l].reshape(1, _H),
                      W2[l], b2[l].reshape(1, _H))
    out2 = _pool_call(h, batf, Wl1, bl1.reshape(1, _H),
                      Wl2, bl2.reshape(1, 1))
    return out2.reshape(-1)


# speedup vs baseline: 4.2963x; 1.0019x over previous
import functools

import jax
import jax.numpy as jnp
from jax import lax
from jax.experimental import pallas as pl
from jax.experimental.pallas import tpu as pltpu
from jax.experimental.pallas import tpu_sc as plsc

_N = 10000
_E = 320000
_H = 128
_ED = 16
_L = 3
_G = 64

_NC = 2
_NS = 16
_NW = _NC * _NS
_EPW = _E // _NW
_K = 40
_NCHUNK = _EPW // _K
_NPAD = 10240
_RPT = _NPAD // _NS


def _sc_body(h_hbm, ec_hbm, src_hbm, dst_hbm, out_hbm,
             aggr_sh, hbuf0, hbuf1, ebuf0, ebuf1, ebuf2, ebuf3,
             sv0, sv1, sv2, sv3, dv0, dv1, dv2, dv3,
             sg0, sg1, se0, se1, se2, se3,
             ss0, ss1, ss2, ss3, si0, si1, si2, si3):
    cid = lax.axis_index("c")
    sid = lax.axis_index("s")
    wid = sid * _NC + cid
    ebase = wid * _EPW

    hbufs = (hbuf0, hbuf1)
    ebufs = (ebuf0, ebuf1, ebuf2, ebuf3)
    srcvs = (sv0, sv1, sv2, sv3)
    dstvs = (dv0, dv1, dv2, dv3)
    sgs = (sg0, sg1)
    ses = (se0, se1, se2, se3)
    sss = (ss0, ss1, ss2, ss3)
    sis = (si0, si1, si2, si3)

    def _zrow(i, _):
        for k in range(_H // 16):
            ebuf0[i, pl.ds(k * 16, 16)] = jnp.zeros((16,), jnp.float32)
        return 0
    lax.fori_loop(0, _K, _zrow, 0)
    row0 = sid * _RPT
    for j in range(_RPT // _K):
        pltpu.sync_copy(ebuf0, aggr_sh.at[pl.ds(row0 + j * _K, _K)])
    plsc.subcore_barrier()

    def _issue_idx(c, q):
        base = ebase + c * _K
        pltpu.async_copy(src_hbm.at[pl.ds(base, _K)], srcvs[q], sis[q])
        pltpu.async_copy(dst_hbm.at[pl.ds(base, _K)], dstvs[q], sis[q])

    def _wait_idx(c, q):
        base = ebase + c * _K
        pltpu.make_async_copy(src_hbm.at[pl.ds(base, _K)], srcvs[q],
                              sis[q]).wait()
        pltpu.make_async_copy(dst_hbm.at[pl.ds(base, _K)], dstvs[q],
                              sis[q]).wait()

    def _issue_gather(slot, q):
        pltpu.async_copy(h_hbm.at[srcvs[q]], hbufs[slot], sgs[slot])

    def _wait_gather(slot, q):
        pltpu.make_async_copy(h_hbm.at[srcvs[q]], hbufs[slot],
                              sgs[slot]).wait()

    def _issue_ec(c, q):
        pltpu.async_copy(ec_hbm.at[pl.ds(ebase + c * _K, _K)],
                         ebufs[q], ses[q])

    def _wait_ec(c, q):
        pltpu.make_async_copy(ec_hbm.at[pl.ds(ebase + c * _K, _K)],
                              ebufs[q], ses[q]).wait()

    def _issue_scatter(q):
        pltpu.async_copy(ebufs[q], aggr_sh.at[dstvs[q]], sss[q], add=True)

    def _wait_scatter(q):
        pltpu.make_async_copy(ebufs[q], aggr_sh.at[dstvs[q]],
                              sss[q]).wait()

    def _compute(slot, q):
        hb, eb = hbufs[slot], ebufs[q]

        def _row(i, _):
            for k in range(_H // 16):
                s = pl.ds(k * 16, 16)
                eb[i, s] = jnp.maximum(hb[i, s] + eb[i, s], 0.0)
            return 0
        lax.fori_loop(0, _K, _row, 0)

    _issue_idx(0, 0)
    _issue_idx(1, 1)
    _wait_idx(0, 0)
    _issue_gather(0, 0)
    _issue_ec(0, 0)
    _issue_ec(1, 1)

    def _step(c, slot, q, traced):
        other, q1, q2, q3 = 1 - slot, (q + 1) % 4, (q + 2) % 4, (q + 3) % 4
        _wait_gather(slot, q)
        _wait_ec(c, q)
        if traced or c + 2 < _NCHUNK:
            _issue_idx(c + 2, q2)
            _issue_ec(c + 2, q2)
        if traced or c + 1 < _NCHUNK:
            _wait_idx(c + 1, q1)
            _issue_gather(other, q1)
        _compute(slot, q)
        if traced:
            @pl.when(c >= 1)
            def _():
                _wait_scatter(q3)
        elif c >= 1:
            _wait_scatter(q3)
        _issue_scatter(q)

    def _quad(t, _):
        c0 = t * 4
        _step(c0 + 0, 0, 0, True)
        _step(c0 + 1, 1, 1, True)
        _step(c0 + 2, 0, 2, True)
        _step(c0 + 3, 1, 3, True)
        return 0
    nloop = (_NCHUNK - 6) // 4
    lax.fori_loop(0, nloop, _quad, 0)
    for c in range(nloop * 4, _NCHUNK):
        _step(c, c % 2, c % 4, False)
    _wait_scatter((_NCHUNK - 1) % 4)
    plsc.subcore_barrier()

    for j in range(_RPT // _K):
        r = row0 + j * _K
        pltpu.sync_copy(aggr_sh.at[pl.ds(r, _K)], ebuf0)
        pltpu.sync_copy(ebuf0, out_hbm.at[cid, pl.ds(r, _K)])


_sc_msg = pl.kernel(
    _sc_body,
    out_type=jax.ShapeDtypeStruct((_NC, _NPAD, _H), jnp.float32),
    mesh=plsc.VectorSubcoreMesh(core_axis_name="c", subcore_axis_name="s"),
    scratch_types=(
        [pltpu.VMEM_SHARED((_NPAD, _H), jnp.float32)]
        + [pltpu.VMEM((_K, _H), jnp.float32) for _ in range(6)]
        + [pltpu.VMEM((_K,), jnp.int32) for _ in range(8)]
        + [pltpu.SemaphoreType.DMA for _ in range(14)]
    ),
)


_BE = 8000


def _ec_body(ea_ref, w_ref, b_ref, out_ref):
    out_ref[...] = jnp.dot(ea_ref[...], w_ref[...],
                           preferred_element_type=jnp.float32) + b_ref[...]


_ec_call = pl.pallas_call(
    _ec_body,
    grid=(_E // _BE,),
    in_specs=[
        pl.BlockSpec((_BE, _ED), lambda i: (i, 0)),
        pl.BlockSpec((_ED, _H), lambda i: (0, 0)),
        pl.BlockSpec((1, _H), lambda i: (0, 0)),
    ],
    out_specs=pl.BlockSpec((_BE, _H), lambda i: (i, 0)),
    out_shape=jax.ShapeDtypeStruct((_E, _H), jnp.float32),
)

_BR = 2000


def _mlp_body(h_ref, p_ref, w1_ref, b1_ref, w2_ref, b2_ref, out_ref):
    z = h_ref[...] + p_ref[0] + p_ref[1]
    t = jnp.maximum(jnp.dot(z, w1_ref[...],
                            preferred_element_type=jnp.float32) + b1_ref[...], 0.0)
    out_ref[...] = jnp.maximum(
        jnp.dot(t, w2_ref[...], preferred_element_type=jnp.float32)
        + b2_ref[...], 0.0)


_mlp_call = pl.pallas_call(
    _mlp_body,
    grid=(_N // _BR,),
    in_specs=[
        pl.BlockSpec((_BR, _H), lambda i: (i, 0)),
        pl.BlockSpec((_NC, _BR, _H), lambda i: (0, i, 0)),
        pl.BlockSpec((_H, _H), lambda i: (0, 0)),
        pl.BlockSpec((1, _H), lambda i: (0, 0)),
        pl.BlockSpec((_H, _H), lambda i: (0, 0)),
        pl.BlockSpec((1, _H), lambda i: (0, 0)),
    ],
    out_specs=pl.BlockSpec((_BR, _H), lambda i: (i, 0)),
    out_shape=jax.ShapeDtypeStruct((_N, _H), jnp.float32),
)


def _pool_body(h_ref, bt_ref, wl1_ref, bl1_ref, wl2_ref, bl2_ref, out_ref):
    bt = bt_ref[...]
    gid = lax.broadcasted_iota(jnp.int32, (_N, _G), 1)
    p = jnp.where(bt == gid, 1.0, 0.0)
    sums = lax.dot_general(p, h_ref[...], (((0,), (0,)), ((), ())),
                           preferred_element_type=jnp.float32)
    counts = jnp.sum(p, axis=0)
    g = sums / jnp.maximum(counts, 1.0)[:, None]
    t = jnp.maximum(jnp.dot(g, wl1_ref[...],
                            preferred_element_type=jnp.float32) + bl1_ref[...],
                    0.0)
    out_ref[...] = (jnp.dot(t, wl2_ref[...],
                            preferred_element_type=jnp.float32) + bl2_ref[...])


_pool_call = pl.pallas_call(
    _pool_body,
    out_shape=jax.ShapeDtypeStruct((_G, 1), jnp.float32),
)


def kernel(x, edge_index, batch, edge_attr, We, be, W1, b1, W2, b2,
           Wl1, bl1, Wl2, bl2):
    src = edge_index[0].astype(jnp.int32)
    dst = edge_index[1].astype(jnp.int32)
    batf = batch.astype(jnp.int32).reshape(_N, 1)
    h = x
    for l in range(_L):
        ec = _ec_call(edge_attr, We[l], be[l].reshape(1, _H))
        parts = _sc_msg(h, ec, src, dst)
        h = _mlp_call(h, parts, W1[l], b1[l].reshape(1, _H),
                      W2[l], b2[l].reshape(1, _H))
    out2 = _pool_call(h, batf, Wl1, bl1.reshape(1, _H),
                      Wl2, bl2.reshape(1, 1))
    return out2.reshape(-1)


# speedup vs baseline: 4.3929x; 1.0225x over previous
import functools

import jax
import jax.numpy as jnp
from jax import lax
from jax.experimental import pallas as pl
from jax.experimental.pallas import tpu as pltpu
from jax.experimental.pallas import tpu_sc as plsc

_N = 10000
_E = 320000
_H = 128
_ED = 16
_L = 3
_G = 64

_NC = 2
_NS = 16
_NW = _NC * _NS
_EPW = _E // _NW
_K = 40
_NCHUNK = _EPW // _K
_NPAD = 10240
_RPT = _NPAD // _NS


def _sc_body(h_hbm, ec_hbm, src_hbm, dst_hbm, out_hbm,
             aggr_sh, hbuf0, hbuf1, ebuf0, ebuf1, ebuf2, ebuf3,
             sv0, sv1, sv2, sv3, dv0, dv1, dv2, dv3,
             sg0, sg1, se0, se1, se2, se3,
             ss0, ss1, ss2, ss3, si0, si1, si2, si3):
    cid = lax.axis_index("c")
    sid = lax.axis_index("s")
    wid = sid * _NC + cid
    ebase = wid * _EPW

    hbufs = (hbuf0, hbuf1)
    ebufs = (ebuf0, ebuf1, ebuf2, ebuf3)
    srcvs = (sv0, sv1, sv2, sv3)
    dstvs = (dv0, dv1, dv2, dv3)
    sgs = (sg0, sg1)
    ses = (se0, se1, se2, se3)
    sss = (ss0, ss1, ss2, ss3)
    sis = (si0, si1, si2, si3)

    def _zrow(i, _):
        for k in range(_H // 16):
            ebuf0[i, pl.ds(k * 16, 16)] = jnp.zeros((16,), jnp.float32)
        return 0
    lax.fori_loop(0, _K, _zrow, 0)
    row0 = sid * _RPT
    for j in range(_RPT // _K):
        pltpu.sync_copy(ebuf0, aggr_sh.at[pl.ds(row0 + j * _K, _K)])
    plsc.subcore_barrier()

    def _issue_idx(c, q):
        base = ebase + c * _K
        pltpu.async_copy(src_hbm.at[pl.ds(base, _K)], srcvs[q], sis[q])
        pltpu.async_copy(dst_hbm.at[pl.ds(base, _K)], dstvs[q], sis[q])

    def _wait_idx(c, q):
        base = ebase + c * _K
        pltpu.make_async_copy(src_hbm.at[pl.ds(base, _K)], srcvs[q],
                              sis[q]).wait()
        pltpu.make_async_copy(dst_hbm.at[pl.ds(base, _K)], dstvs[q],
                              sis[q]).wait()

    def _issue_gather(slot, q):
        pltpu.async_copy(h_hbm.at[srcvs[q]], hbufs[slot], sgs[slot])

    def _wait_gather(slot, q):
        pltpu.make_async_copy(h_hbm.at[srcvs[q]], hbufs[slot],
                              sgs[slot]).wait()

    def _issue_ec(c, q):
        pltpu.async_copy(ec_hbm.at[pl.ds(ebase + c * _K, _K)],
                         ebufs[q], ses[q])

    def _wait_ec(c, q):
        pltpu.make_async_copy(ec_hbm.at[pl.ds(ebase + c * _K, _K)],
                              ebufs[q], ses[q]).wait()

    def _issue_scatter(q):
        pltpu.async_copy(ebufs[q], aggr_sh.at[dstvs[q]], sss[q], add=True)

    def _wait_scatter(q):
        pltpu.make_async_copy(ebufs[q], aggr_sh.at[dstvs[q]],
                              sss[q]).wait()

    def _compute(slot, q):
        hb, eb = hbufs[slot], ebufs[q]

        def _row(i, _):
            for k in range(_H // 16):
                s = pl.ds(k * 16, 16)
                eb[i, s] = jnp.maximum(hb[i, s] + eb[i, s], 0.0)
            return 0
        lax.fori_loop(0, _K, _row, 0)

    _issue_idx(0, 0)
    _issue_idx(1, 1)
    _wait_idx(0, 0)
    _issue_gather(0, 0)
    _issue_ec(0, 0)
    _issue_ec(1, 1)

    def _step(c, slot, q, traced):
        other, q1, q2, q3 = 1 - slot, (q + 1) % 4, (q + 2) % 4, (q + 3) % 4
        _wait_gather(slot, q)
        _wait_ec(c, q)
        if traced or c + 1 < _NCHUNK:
            _wait_idx(c + 1, q1)
            _issue_gather(other, q1)
        _compute(slot, q)
        if traced or c + 2 < _NCHUNK:
            _issue_idx(c + 2, q2)
            _issue_ec(c + 2, q2)
        if traced:
            @pl.when(c >= 1)
            def _():
                _wait_scatter(q3)
        elif c >= 1:
            _wait_scatter(q3)
        _issue_scatter(q)

    def _quad(t, _):
        c0 = t * 4
        _step(c0 + 0, 0, 0, True)
        _step(c0 + 1, 1, 1, True)
        _step(c0 + 2, 0, 2, True)
        _step(c0 + 3, 1, 3, True)
        return 0
    nloop = (_NCHUNK - 6) // 4
    lax.fori_loop(0, nloop, _quad, 0)
    for c in range(nloop * 4, _NCHUNK):
        _step(c, c % 2, c % 4, False)
    _wait_scatter((_NCHUNK - 1) % 4)
    plsc.subcore_barrier()

    for j in range(_RPT // _K):
        r = row0 + j * _K
        pltpu.sync_copy(aggr_sh.at[pl.ds(r, _K)], ebuf0)
        pltpu.sync_copy(ebuf0, out_hbm.at[cid, pl.ds(r, _K)])


_sc_msg = pl.kernel(
    _sc_body,
    out_type=jax.ShapeDtypeStruct((_NC, _NPAD, _H), jnp.float32),
    mesh=plsc.VectorSubcoreMesh(core_axis_name="c", subcore_axis_name="s"),
    scratch_types=(
        [pltpu.VMEM_SHARED((_NPAD, _H), jnp.float32)]
        + [pltpu.VMEM((_K, _H), jnp.float32) for _ in range(6)]
        + [pltpu.VMEM((_K,), jnp.int32) for _ in range(8)]
        + [pltpu.SemaphoreType.DMA for _ in range(14)]
    ),
)


_BE = 8000


def _ec_body(ea_ref, w_ref, b_ref, out_ref):
    out_ref[...] = jnp.dot(ea_ref[...], w_ref[...],
                           preferred_element_type=jnp.float32) + b_ref[...]


_ec_call = pl.pallas_call(
    _ec_body,
    grid=(_E // _BE,),
    in_specs=[
        pl.BlockSpec((_BE, _ED), lambda i: (i, 0)),
        pl.BlockSpec((_ED, _H), lambda i: (0, 0)),
        pl.BlockSpec((1, _H), lambda i: (0, 0)),
    ],
    out_specs=pl.BlockSpec((_BE, _H), lambda i: (i, 0)),
    out_shape=jax.ShapeDtypeStruct((_E, _H), jnp.float32),
)

_BR = 2000


def _mlp_body(h_ref, p_ref, w1_ref, b1_ref, w2_ref, b2_ref, out_ref):
    z = h_ref[...] + p_ref[0] + p_ref[1]
    t = jnp.maximum(jnp.dot(z, w1_ref[...],
                            preferred_element_type=jnp.float32) + b1_ref[...], 0.0)
    out_ref[...] = jnp.maximum(
        jnp.dot(t, w2_ref[...], preferred_element_type=jnp.float32)
        + b2_ref[...], 0.0)


_mlp_call = pl.pallas_call(
    _mlp_body,
    grid=(_N // _BR,),
    in_specs=[
        pl.BlockSpec((_BR, _H), lambda i: (i, 0)),
        pl.BlockSpec((_NC, _BR, _H), lambda i: (0, i, 0)),
        pl.BlockSpec((_H, _H), lambda i: (0, 0)),
        pl.BlockSpec((1, _H), lambda i: (0, 0)),
        pl.BlockSpec((_H, _H), lambda i: (0, 0)),
        pl.BlockSpec((1, _H), lambda i: (0, 0)),
    ],
    out_specs=pl.BlockSpec((_BR, _H), lambda i: (i, 0)),
    out_shape=jax.ShapeDtypeStruct((_N, _H), jnp.float32),
)


def _pool_body(h_ref, bt_ref, wl1_ref, bl1_ref, wl2_ref, bl2_ref, out_ref):
    bt = bt_ref[...]
    gid = lax.broadcasted_iota(jnp.int32, (_N, _G), 1)
    p = jnp.where(bt == gid, 1.0, 0.0)
    sums = lax.dot_general(p, h_ref[...], (((0,), (0,)), ((), ())),
                           preferred_element_type=jnp.float32,
                           precision=lax.Precision.HIGHEST)
    counts = jnp.sum(p, axis=0)
    g = sums / jnp.maximum(counts, 1.0)[:, None]
    t = jnp.maximum(jnp.dot(g, wl1_ref[...],
                            preferred_element_type=jnp.float32) + bl1_ref[...],
                    0.0)
    out_ref[...] = (jnp.dot(t, wl2_ref[...],
                            preferred_element_type=jnp.float32) + bl2_ref[...])


_pool_call = pl.pallas_call(
    _pool_body,
    out_shape=jax.ShapeDtypeStruct((_G, 1), jnp.float32),
)


def kernel(x, edge_index, batch, edge_attr, We, be, W1, b1, W2, b2,
           Wl1, bl1, Wl2, bl2):
    src = edge_index[0].astype(jnp.int32)
    dst = edge_index[1].astype(jnp.int32)
    batf = batch.astype(jnp.int32).reshape(_N, 1)
    h = x
    for l in range(_L):
        ec = _ec_call(edge_attr, We[l], be[l].reshape(1, _H))
        parts = _sc_msg(h, ec, src, dst)
        h = _mlp_call(h, parts, W1[l], b1[l].reshape(1, _H),
                      W2[l], b2[l].reshape(1, _H))
    out2 = _pool_call(h, batf, Wl1, bl1.reshape(1, _H),
                      Wl2, bl2.reshape(1, 1))
    return out2.reshape(-1)
